# baseline (speedup 1.0000x reference)
import functools

import jax
import jax.numpy as jnp
from jax import lax
from jax.experimental import pallas as pl
from jax.experimental.pallas import tpu as pltpu
from jax.experimental.pallas import tpu_sc as plsc

_B = 16384
_D = 64
_NC = 2
_NS = 16
_NW = _NC * _NS
_BPW = _B // _NW


def _make_sc_gather():
  mesh = plsc.VectorSubcoreMesh(core_axis_name="c", subcore_axis_name="s")

  @functools.partial(
      pl.kernel,
      out_type=(
          jax.ShapeDtypeStruct((_B, _D), jnp.float32),
          jax.ShapeDtypeStruct((_B, _D), jnp.float32),
      ),
      mesh=mesh,
      compiler_params=pltpu.CompilerParams(use_tc_tiling_on_sc=False),
      scratch_types=[
          pltpu.VMEM((_BPW,), jnp.int32),
          pltpu.VMEM((_BPW, _D), jnp.float32),
          pltpu.VMEM((_BPW,), jnp.int32),
          pltpu.VMEM((_BPW, _D), jnp.float32),
          pltpu.SemaphoreType.DMA,
          pltpu.SemaphoreType.DMA,
      ],
  )
  def sc_gather(u_hbm, i_hbm, utab_hbm, itab_hbm, out_u, out_i,
                uidx_v, urows_v, iidx_v, irows_v, sem_u, sem_i):
    wid = lax.axis_index("s") * _NC + lax.axis_index("c")
    base = wid * _BPW
    pltpu.sync_copy(u_hbm.at[pl.ds(base, _BPW)], uidx_v)
    pltpu.sync_copy(i_hbm.at[pl.ds(base, _BPW)], iidx_v)
    cu = pltpu.async_copy(utab_hbm.at[uidx_v], urows_v, sem_u)
    ci = pltpu.async_copy(itab_hbm.at[iidx_v], irows_v, sem_i)
    cu.wait()
    ci.wait()
    pltpu.sync_copy(urows_v, out_u.at[pl.ds(base, _BPW)])
    pltpu.sync_copy(irows_v, out_i.at[pl.ds(base, _BPW)])

  return sc_gather


def _dense_body(gu, gi, ua, ia, wut, wit, w1u, w1i, w2, bu, bi, b1, b2, out):
  hp = jax.lax.Precision.HIGHEST
  u_e = gu[...] + jnp.dot(ua[...], wut[...], precision=hp,
                          preferred_element_type=jnp.float32) + bu[...]
  i_e = gi[...] + jnp.dot(ia[...], wit[...], precision=hp,
                          preferred_element_type=jnp.float32) + bi[...]
  h = jnp.dot(u_e, w1u[...], precision=hp, preferred_element_type=jnp.float32)
  h = h + jnp.dot(i_e, w1i[...], precision=hp,
                  preferred_element_type=jnp.float32)
  h = jnp.maximum(h + b1[...], 0.0)
  out[...] = jnp.dot(h, w2[...], precision=hp,
                     preferred_element_type=jnp.float32) + b2[...]


def _make_dense(bm):
  grid = _B // bm
  full = lambda r, c: pl.BlockSpec((r, c), lambda j: (0, 0))
  return pl.pallas_call(
      _dense_body,
      grid=(grid,),
      in_specs=[
          pl.BlockSpec((bm, _D), lambda j: (j, 0)),
          pl.BlockSpec((bm, _D), lambda j: (j, 0)),
          pl.BlockSpec((bm, 128), lambda j: (j, 0)),
          pl.BlockSpec((bm, 128), lambda j: (j, 0)),
          full(128, _D),
          full(128, _D),
          full(_D, 128),
          full(_D, 128),
          full(128, 1),
          full(1, _D),
          full(1, _D),
          full(1, 128),
          full(1, 1),
      ],
      out_specs=pl.BlockSpec((bm, 1), lambda j: (j, 0)),
      out_shape=jax.ShapeDtypeStruct((_B, 1), jnp.float32),
  )


_make_sc_gather = functools.cache(_make_sc_gather)
_make_dense = functools.cache(_make_dense)


def kernel(u, i, ua, ia, user_emb, item_emb, Wu, bu, Wi, bi, W1, b1, W2, b2):
  gu, gi = _make_sc_gather()(u, i, user_emb, item_emb)
  _dense = _make_dense(2048)
  w1t = W1.T
  out = _dense(
      gu, gi, ua, ia,
      Wu.T, Wi.T, w1t[:_D, :], w1t[_D:, :], W2.T,
      bu.reshape(1, _D), bi.reshape(1, _D),
      b1.reshape(1, 128), b2.reshape(1, 1),
  )
  return out[:, 0]
